# baseline (speedup 1.0000x reference)
import functools

import jax
import jax.numpy as jnp
from jax import lax
from jax.experimental import pallas as pl
from jax.experimental.pallas import tpu as pltpu
from jax.experimental.pallas import tpu_sc as plsc

_NUM_GRAPHS = 16
_RATIO = 0.8
_NC = 2
_NS = 16
_NW = _NC * _NS


def _emb_gather(table, idx):
    B = idx.shape[0]
    D = table.shape[1]
    assert B % (8 * _NW) == 0
    b_per_w = B // _NW
    mesh = plsc.VectorSubcoreMesh(core_axis_name="c", subcore_axis_name="s")

    @functools.partial(
        pl.kernel,
        mesh=mesh,
        out_type=jax.ShapeDtypeStruct((B, D), jnp.float32),
        scratch_types=[
            pltpu.VMEM((b_per_w,), jnp.int32),
            pltpu.VMEM((b_per_w, D), jnp.float32),
            pltpu.SemaphoreType.DMA,
        ],
    )
    def k(table_hbm, idx_hbm, out_hbm, idx_v, rows_v, sem):
        wid = lax.axis_index("s") * _NC + lax.axis_index("c")
        base = wid * b_per_w
        pltpu.sync_copy(idx_hbm.at[pl.ds(base, b_per_w)], idx_v)
        pltpu.async_copy(table_hbm.at[idx_v], rows_v, sem).wait()
        pltpu.sync_copy(rows_v, out_hbm.at[pl.ds(base, b_per_w)])

    return k(table, idx)


def _gcn_conv(x, src, dst, edge_mask, node_mask, W, b):
    N = x.shape[0]
    em = edge_mask.astype(x.dtype)
    nm = node_mask.astype(x.dtype)
    deg = jnp.zeros(N, x.dtype).at[dst].add(em) + nm
    dinv = jnp.where(deg > 0, 1.0 / jnp.sqrt(jnp.maximum(deg, 1e-12)), 0.0)
    xw = x @ W
    norm = dinv[src] * dinv[dst] * em
    out = jnp.zeros_like(xw).at[dst].add(xw[src] * norm[:, None])
    out = out + xw * (dinv * dinv * nm)[:, None]
    return (out + b) * nm[:, None]


def _topk_pool(x, node_mask, src, dst, edge_mask, batch, p):
    N = x.shape[0]
    score = jnp.tanh((x @ p) / jnp.linalg.norm(p))
    score_m = jnp.where(node_mask, score, -1e9)
    counts = jax.ops.segment_sum(node_mask.astype(jnp.int32), batch,
                                 num_segments=_NUM_GRAPHS)
    k = jnp.ceil(_RATIO * counts.astype(jnp.float32)).astype(jnp.int32)
    total = jax.ops.segment_sum(jnp.ones(N, jnp.int32), batch,
                                num_segments=_NUM_GRAPHS)
    starts = jnp.concatenate(
        [jnp.zeros(1, jnp.int32), jnp.cumsum(total)[:-1].astype(jnp.int32)])
    order = jnp.lexsort((-score_m, batch))
    pos = jnp.arange(N, dtype=jnp.int32)
    rank = jnp.zeros(N, jnp.int32).at[order].set(pos - starts[batch[order]])
    keep = node_mask & (rank < k[batch])
    x_new = jnp.where(keep[:, None], x * score[:, None], 0.0)
    edge_mask_new = edge_mask & keep[src] & keep[dst]
    return x_new, keep, edge_mask_new


def _gmp(x, batch, node_mask):
    xm = jnp.where(node_mask[:, None], x, -1e9)
    mx = jax.ops.segment_max(xm, batch, num_segments=_NUM_GRAPHS)
    cnt = jax.ops.segment_sum(node_mask.astype(x.dtype), batch,
                              num_segments=_NUM_GRAPHS)
    return jnp.where(cnt[:, None] > 0, mx, 0.0)


def _gap(x, batch, node_mask):
    s = jax.ops.segment_sum(jnp.where(node_mask[:, None], x, 0.0), batch,
                            num_segments=_NUM_GRAPHS)
    cnt = jax.ops.segment_sum(node_mask.astype(x.dtype), batch,
                              num_segments=_NUM_GRAPHS)
    return s / jnp.maximum(cnt, 1.0)[:, None]


def kernel(x, edge_index, batch, emb, W1, b1, p1, W2, b2, p2, W3, b3, p3,
           lw1, lb1, lw2, lb2, lw3, lb3):
    src, dst = edge_index[0], edge_index[1]
    N = x.shape[0]
    B = ((N + 255) // 256) * 256
    idx = jnp.concatenate([x[:, 0], jnp.zeros(B - N, jnp.int32)])
    h = _emb_gather(emb, idx)[:N]
    node_mask = jnp.ones(N, bool)
    edge_mask = jnp.ones(src.shape[0], bool)
    h = jax.nn.relu(_gcn_conv(h, src, dst, edge_mask, node_mask, W1, b1))
    h, node_mask, edge_mask = _topk_pool(h, node_mask, src, dst, edge_mask, batch, p1)
    x1 = jnp.concatenate([_gmp(h, batch, node_mask), _gap(h, batch, node_mask)], axis=1)
    h = jax.nn.relu(_gcn_conv(h, src, dst, edge_mask, node_mask, W2, b2))
    h, node_mask, edge_mask = _topk_pool(h, node_mask, src, dst, edge_mask, batch, p2)
    x2 = jnp.concatenate([_gmp(h, batch, node_mask), _gap(h, batch, node_mask)], axis=1)
    h = jax.nn.relu(_gcn_conv(h, src, dst, edge_mask, node_mask, W3, b3))
    h, node_mask, edge_mask = _topk_pool(h, node_mask, src, dst, edge_mask, batch, p3)
    x3 = jnp.concatenate([_gmp(h, batch, node_mask), _gap(h, batch, node_mask)], axis=1)
    z = x1 + x2 + x3
    z = jax.nn.relu(z @ lw1 + lb1)
    z = jax.nn.relu(z @ lw2 + lb2)
    out = jax.nn.sigmoid(z @ lw3 + lb3)[:, 0]
    return out
